# baseline (speedup 1.0000x reference)
import functools

import jax
import jax.numpy as jnp
from jax import lax
from jax.experimental import pallas as pl
from jax.experimental.pallas import tpu as pltpu
from jax.experimental.pallas import tpu_sc as plsc

_B, _N, _F, _D = 4, 100000, 200000, 128
_NC, _NS = 2, 16
_NW = _NC * _NS
_WPB = _NW // _B
_FPW = _F // _WPB
_NCH = 5
_CH = _FPW // _NCH
_CHP = ((_CH + 15) // 16) * 16
_GSTEPS = _CHP // 16


def _edges_body(face_hbm, out_hbm, faces_hbm, fin, cols):
    wid = lax.axis_index("s") * _NC + lax.axis_index("c")
    b = wid // _WPB
    i0 = (wid % _WPB) * _FPW
    iota3 = lax.iota(jnp.int32, 16) * 3

    def chunk_body(ch, carry):
        base = i0 + ch * _CH
        fbase = b * 3 * _F + 3 * base
        pltpu.sync_copy(face_hbm.at[pl.ds(fbase, 3 * _CH)],
                        fin.at[pl.ds(0, 3 * _CH)])
        pltpu.sync_copy(fin.at[pl.ds(0, 3 * _CH)],
                        faces_hbm.at[pl.ds(fbase, 3 * _CH)])

        def gather_body(j, c2):
            src = j * 48 + iota3
            dst = j * 16
            for c in range(3):
                vals = plsc.load_gather(fin, [src + c])
                cols[pl.ds(c * _CHP + dst, 16)] = vals
            return c2

        lax.fori_loop(0, _GSTEPS, gather_body, 0, unroll=2)

        obase = b * 6 * _F + base
        for c in range(3):
            col = cols.at[pl.ds(c * _CHP, _CH)]
            pltpu.sync_copy(col, out_hbm.at[pl.ds(obase + c * _F, _CH)])
            s1 = (c + 2) % 3
            pltpu.sync_copy(col, out_hbm.at[pl.ds(obase + 3 * _F + s1 * _F, _CH)])
        return carry

    lax.fori_loop(0, _NCH, chunk_body, 0)


_edges_call = functools.partial(
    pl.kernel,
    mesh=plsc.VectorSubcoreMesh(core_axis_name="c", subcore_axis_name="s"),
    out_type=(
        jax.ShapeDtypeStruct((_B * 2 * 3 * _F,), jnp.int32),
        jax.ShapeDtypeStruct((_B * 3 * _F,), jnp.int32),
    ),
    compiler_params=pltpu.CompilerParams(needs_layout_passes=False),
    scratch_types=[
        pltpu.VMEM((3 * _CH + 64,), jnp.int32),
        pltpu.VMEM((3 * _CHP,), jnp.int32),
    ],
)(_edges_body)


_ROWS = _B * _N
_RBLK = 8000


def _copy_body(src_ref, dst_ref):
    dst_ref[...] = src_ref[...]


def _verts_copy(x2):
    return pl.pallas_call(
        _copy_body,
        grid=(_ROWS // _RBLK,),
        in_specs=[pl.BlockSpec((_RBLK, _D), lambda i: (i, 0))],
        out_specs=pl.BlockSpec((_RBLK, _D), lambda i: (i, 0)),
        out_shape=jax.ShapeDtypeStruct((_ROWS, _D), jnp.float32),
    )(x2)


def kernel(x, mask, face):
    del mask
    face_flat = face.reshape(_B * 3 * _F)
    edges_flat, faces_flat = _edges_call(face_flat)
    new_edges = edges_flat.reshape(_B, 2, 3 * _F)
    new_faces = faces_flat.reshape(_B, _F, 3)
    new_verts = _verts_copy(x.reshape(_ROWS, _D)).reshape(_B, _N, _D)
    return (new_verts, new_faces, new_edges)


# speedup vs baseline: 24.5472x; 24.5472x over previous
import jax
import jax.numpy as jnp
from jax.experimental import pallas as pl

_B, _N, _F, _D = 4, 100000, 200000, 128


def _edges_body(face_ref, out_ref):
    b = pl.program_id(0)
    for r in range(2):
        for s in range(3):
            c = s if r == 0 else (s + 1) % 3
            out_ref[0, r, pl.ds(s * _F, _F)] = face_ref[c, b, :]


def _edges_call(face_t):
    return pl.pallas_call(
        _edges_body,
        grid=(_B,),
        in_specs=[pl.BlockSpec((3, _B, _F), lambda b: (0, 0, 0))],
        out_specs=pl.BlockSpec((1, 2, 3 * _F), lambda b: (b, 0, 0)),
        out_shape=jax.ShapeDtypeStruct((_B, 2, 3 * _F), jnp.int32),
    )(face_t)


def _copy_body(src_ref, dst_ref):
    dst_ref[...] = src_ref[...]


def _faces_call(face_t):
    return pl.pallas_call(
        _copy_body,
        grid=(3,),
        in_specs=[pl.BlockSpec((1, _B, _F), lambda c: (c, 0, 0))],
        out_specs=pl.BlockSpec((1, _B, _F), lambda c: (c, 0, 0)),
        out_shape=jax.ShapeDtypeStruct((3, _B, _F), jnp.int32),
    )(face_t)


_ROWS = _B * _N
_RBLK = 8000


def _verts_copy(x2):
    return pl.pallas_call(
        _copy_body,
        grid=(_ROWS // _RBLK,),
        in_specs=[pl.BlockSpec((_RBLK, _D), lambda i: (i, 0))],
        out_specs=pl.BlockSpec((_RBLK, _D), lambda i: (i, 0)),
        out_shape=jax.ShapeDtypeStruct((_ROWS, _D), jnp.float32),
    )(x2)


def kernel(x, mask, face):
    del mask
    face_t = jnp.transpose(face, (2, 0, 1))
    new_edges = _edges_call(face_t)
    new_faces = jnp.transpose(_faces_call(face_t), (1, 2, 0))
    new_verts = _verts_copy(x.reshape(_ROWS, _D)).reshape(_B, _N, _D)
    return (new_verts, new_faces, new_edges)
